# Initial kernel scaffold; baseline (speedup 1.0000x reference)
import jax
import jax.numpy as jnp
from jax.experimental import pallas as pl


def kernel(x, emb_0, emb_1, emb_2, emb_3, emb_4, emb_5, emb_6, emb_7, emb_8, W, b):
    raise NotImplementedError("write your pallas kernel here")


# speedup vs baseline: 7.8754x; 7.8754x over previous
import functools
import numpy as np
import jax
import jax.numpy as jnp
from jax.experimental import pallas as pl
from jax.experimental.pallas import tpu as pltpu

_FEATURE_DIMS = [119, 5, 12, 12, 10, 6, 6, 2, 2]
_OFFS = [0, 119, 124, 136, 148, 158, 164, 170, 172]
_NCAT = 9
_TOT = 174
_VPAD = 256
_EMB = 64
_BLOCK = 2000


def _body(x_ref, e_ref, w_ref, b_ref, o_ref):
    xb = x_ref[...]
    idx = xb[:, :_NCAT].astype(jnp.int32)
    cols = jax.lax.broadcasted_iota(jnp.int32, (_BLOCK, _VPAD), 1)
    oh = jnp.zeros((_BLOCK, _VPAD), jnp.float32)
    for i in range(_NCAT):
        code = idx[:, i][:, None] + _OFFS[i]
        oh = oh + (code == cols).astype(jnp.float32)
    emb = jnp.dot(oh, e_ref[...], preferred_element_type=jnp.float32)
    sig = xb[:, _NCAT:]
    out = emb @ w_ref[:_EMB, :] + sig @ w_ref[_EMB:, :] + b_ref[...]
    o_ref[...] = out


@jax.jit
def kernel(x, emb_0, emb_1, emb_2, emb_3, emb_4, emb_5, emb_6, emb_7, emb_8, W, b):
    n = x.shape[0]
    tables = [emb_0, emb_1, emb_2, emb_3, emb_4, emb_5, emb_6, emb_7, emb_8]
    e = jnp.concatenate(
        tables + [jnp.zeros((_VPAD - _TOT, _EMB), jnp.float32)], axis=0
    )
    b2 = b.reshape(1, _EMB)
    grid = (n // _BLOCK,)
    return pl.pallas_call(
        _body,
        grid=grid,
        in_specs=[
            pl.BlockSpec((_BLOCK, x.shape[1]), lambda i: (i, 0)),
            pl.BlockSpec((_VPAD, _EMB), lambda i: (0, 0)),
            pl.BlockSpec((W.shape[0], _EMB), lambda i: (0, 0)),
            pl.BlockSpec((1, _EMB), lambda i: (0, 0)),
        ],
        out_specs=pl.BlockSpec((_BLOCK, _EMB), lambda i: (i, 0)),
        out_shape=jax.ShapeDtypeStruct((n, _EMB), jnp.float32),
        compiler_params=pltpu.CompilerParams(
            dimension_semantics=("arbitrary",),
        ),
    )(x, e, W, b2)


# speedup vs baseline: 12.2731x; 1.5584x over previous
import numpy as np
import jax
import jax.numpy as jnp
from jax.experimental import pallas as pl
from jax.experimental.pallas import tpu as pltpu

_FEATURE_DIMS = [119, 5, 12, 12, 10, 6, 6, 2, 2]
_OFFS = [0, 119, 124, 136, 148, 158, 164, 170, 172]
_NCAT = 9
_TOT = 174
_VPAD = 256
_EMB = 64
_NCOL = _NCAT + 32
_BLOCK = 2000

_S = np.zeros((_NCOL, _VPAD), np.float32)
_T = np.full((1, _VPAD), -1.0, np.float32)
for _i in range(_NCAT):
    _lo = _OFFS[_i]
    _hi = _lo + _FEATURE_DIMS[_i]
    _S[_i, _lo:_hi] = 1.0
    _T[0, _lo:_hi] = np.arange(_hi - _lo, dtype=np.float32)


def _body(x_ref, s_ref, t_ref, e_ref, w_ref, b_ref, o_ref):
    xb = x_ref[...]
    c = jnp.dot(jnp.trunc(xb), s_ref[...], preferred_element_type=jnp.float32)
    oh = (c == t_ref[...]).astype(jnp.float32)
    emb = jnp.dot(oh, e_ref[...], preferred_element_type=jnp.float32)
    sig = xb[:, _NCAT:]
    out = emb @ w_ref[:_EMB, :] + sig @ w_ref[_EMB:, :] + b_ref[...]
    o_ref[...] = out


@jax.jit
def kernel(x, emb_0, emb_1, emb_2, emb_3, emb_4, emb_5, emb_6, emb_7, emb_8, W, b):
    n = x.shape[0]
    tables = [emb_0, emb_1, emb_2, emb_3, emb_4, emb_5, emb_6, emb_7, emb_8]
    e = jnp.concatenate(
        tables + [jnp.zeros((_VPAD - _TOT, _EMB), jnp.float32)], axis=0
    )
    s = jnp.asarray(_S)
    t = jnp.asarray(_T)
    b2 = b.reshape(1, _EMB)
    grid = (n // _BLOCK,)
    return pl.pallas_call(
        _body,
        grid=grid,
        in_specs=[
            pl.BlockSpec((_BLOCK, _NCOL), lambda i: (i, 0)),
            pl.BlockSpec((_NCOL, _VPAD), lambda i: (0, 0)),
            pl.BlockSpec((1, _VPAD), lambda i: (0, 0)),
            pl.BlockSpec((_VPAD, _EMB), lambda i: (0, 0)),
            pl.BlockSpec((W.shape[0], _EMB), lambda i: (0, 0)),
            pl.BlockSpec((1, _EMB), lambda i: (0, 0)),
        ],
        out_specs=pl.BlockSpec((_BLOCK, _EMB), lambda i: (i, 0)),
        out_shape=jax.ShapeDtypeStruct((n, _EMB), jnp.float32),
        compiler_params=pltpu.CompilerParams(
            dimension_semantics=("arbitrary",),
        ),
    )(x, s, t, e, W, b2)


# speedup vs baseline: 13.6253x; 1.1102x over previous
import numpy as np
import jax
import jax.numpy as jnp
from jax.experimental import pallas as pl
from jax.experimental.pallas import tpu as pltpu

_FEATURE_DIMS = [119, 5, 12, 12, 10, 6, 6, 2, 2]
_OFFS = [0, 119, 124, 136, 148, 158, 164, 170, 172]
_NCAT = 9
_TOT = 174
_VPAD = 256
_EMB = 64
_NCOL = _NCAT + 32
_BLOCK = 4000

_S = np.zeros((_NCOL, _VPAD), np.float32)
_T = np.full((1, _VPAD), -1.0, np.float32)
for _i in range(_NCAT):
    _lo = _OFFS[_i]
    _hi = _lo + _FEATURE_DIMS[_i]
    _S[_i, _lo:_hi] = 1.0
    _T[0, _lo:_hi] = np.arange(_hi - _lo, dtype=np.float32)


def _body(x_ref, s_ref, t_ref, e_ref, w_ref, b_ref, o_ref):
    xb = x_ref[...]
    c = jnp.dot(jnp.trunc(xb), s_ref[...], preferred_element_type=jnp.float32)
    oh = (c == t_ref[...]).astype(jnp.float32)
    emb = jnp.dot(oh, e_ref[...], preferred_element_type=jnp.float32)
    sig = xb[:, _NCAT:]
    out = emb @ w_ref[:_EMB, :] + sig @ w_ref[_EMB:, :] + b_ref[...]
    o_ref[...] = out


@jax.jit
def kernel(x, emb_0, emb_1, emb_2, emb_3, emb_4, emb_5, emb_6, emb_7, emb_8, W, b):
    n = x.shape[0]
    tables = [emb_0, emb_1, emb_2, emb_3, emb_4, emb_5, emb_6, emb_7, emb_8]
    e = jnp.concatenate(
        tables + [jnp.zeros((_VPAD - _TOT, _EMB), jnp.float32)], axis=0
    )
    s = jnp.asarray(_S)
    t = jnp.asarray(_T)
    b2 = b.reshape(1, _EMB)
    grid = (n // _BLOCK,)
    return pl.pallas_call(
        _body,
        grid=grid,
        in_specs=[
            pl.BlockSpec((_BLOCK, _NCOL), lambda i: (i, 0)),
            pl.BlockSpec((_NCOL, _VPAD), lambda i: (0, 0)),
            pl.BlockSpec((1, _VPAD), lambda i: (0, 0)),
            pl.BlockSpec((_VPAD, _EMB), lambda i: (0, 0)),
            pl.BlockSpec((W.shape[0], _EMB), lambda i: (0, 0)),
            pl.BlockSpec((1, _EMB), lambda i: (0, 0)),
        ],
        out_specs=pl.BlockSpec((_BLOCK, _EMB), lambda i: (i, 0)),
        out_shape=jax.ShapeDtypeStruct((n, _EMB), jnp.float32),
        compiler_params=pltpu.CompilerParams(
            dimension_semantics=("arbitrary",),
        ),
    )(x, s, t, e, W, b2)


# speedup vs baseline: 14.4574x; 1.0611x over previous
import numpy as np
import jax
import jax.numpy as jnp
from jax.experimental import pallas as pl
from jax.experimental.pallas import tpu as pltpu

_FEATURE_DIMS = [119, 5, 12, 12, 10, 6, 6, 2, 2]
_OFFS = [0, 119, 124, 136, 148, 158, 164, 170, 172]
_NCAT = 9
_TOT = 174
_VPAD = 256
_EMB = 64
_NCOL = _NCAT + 32
_BLOCK = 10000

_S = np.zeros((_NCOL, _VPAD), np.float32)
_T = np.full((1, _VPAD), -1.0, np.float32)
for _i in range(_NCAT):
    _lo = _OFFS[_i]
    _hi = _lo + _FEATURE_DIMS[_i]
    _S[_i, _lo:_hi] = 1.0
    _T[0, _lo:_hi] = np.arange(_hi - _lo, dtype=np.float32)


def _body(x_ref, s_ref, t_ref, e_ref, w_ref, b_ref, o_ref):
    xb = x_ref[...]
    c = jnp.dot(jnp.trunc(xb), s_ref[...], preferred_element_type=jnp.float32)
    oh = (c == t_ref[...]).astype(jnp.float32)
    emb = jnp.dot(oh, e_ref[...], preferred_element_type=jnp.float32)
    sig = xb[:, _NCAT:]
    out = emb @ w_ref[:_EMB, :] + sig @ w_ref[_EMB:, :] + b_ref[...]
    o_ref[...] = out


@jax.jit
def kernel(x, emb_0, emb_1, emb_2, emb_3, emb_4, emb_5, emb_6, emb_7, emb_8, W, b):
    n = x.shape[0]
    tables = [emb_0, emb_1, emb_2, emb_3, emb_4, emb_5, emb_6, emb_7, emb_8]
    e = jnp.concatenate(
        tables + [jnp.zeros((_VPAD - _TOT, _EMB), jnp.float32)], axis=0
    )
    s = jnp.asarray(_S)
    t = jnp.asarray(_T)
    b2 = b.reshape(1, _EMB)
    grid = (n // _BLOCK,)
    return pl.pallas_call(
        _body,
        grid=grid,
        in_specs=[
            pl.BlockSpec((_BLOCK, _NCOL), lambda i: (i, 0)),
            pl.BlockSpec((_NCOL, _VPAD), lambda i: (0, 0)),
            pl.BlockSpec((1, _VPAD), lambda i: (0, 0)),
            pl.BlockSpec((_VPAD, _EMB), lambda i: (0, 0)),
            pl.BlockSpec((W.shape[0], _EMB), lambda i: (0, 0)),
            pl.BlockSpec((1, _EMB), lambda i: (0, 0)),
        ],
        out_specs=pl.BlockSpec((_BLOCK, _EMB), lambda i: (i, 0)),
        out_shape=jax.ShapeDtypeStruct((n, _EMB), jnp.float32),
        compiler_params=pltpu.CompilerParams(
            dimension_semantics=("arbitrary",),
        ),
    )(x, s, t, e, W, b2)


# speedup vs baseline: 42.4476x; 2.9360x over previous
import numpy as np
import jax
import jax.numpy as jnp
from jax.experimental import pallas as pl
from jax.experimental.pallas import tpu as pltpu

_FEATURE_DIMS = [119, 5, 12, 12, 10, 6, 6, 2, 2]
_OFFS = [0, 119, 124, 136, 148, 158, 164, 170, 172]
_NCAT = 9
_TOT = 174
_VPAD = 256
_EMB = 64
_NCOL = _NCAT + 32
_BLOCK = 8192

_S = np.zeros((_VPAD, _NCAT), np.float32)
_T = np.full((_VPAD, 1), -1.0, np.float32)
for _i in range(_NCAT):
    _lo = _OFFS[_i]
    _hi = _lo + _FEATURE_DIMS[_i]
    _S[_lo:_hi, _i] = 1.0
    _T[_lo:_hi, 0] = np.arange(_hi - _lo, dtype=np.float32)


def _body(xt_ref, st_ref, tt_ref, et_ref, w1t_ref, w2t_ref, b_ref, o_ref):
    xt = xt_ref[...]
    cat = jnp.trunc(xt[:_NCAT, :])
    c = jnp.dot(st_ref[...], cat, preferred_element_type=jnp.float32)
    oh = (c == tt_ref[...]).astype(jnp.float32)
    a = jnp.dot(w1t_ref[...], et_ref[...], preferred_element_type=jnp.float32)
    emb = jnp.dot(a, oh, preferred_element_type=jnp.float32)
    sig = jnp.dot(w2t_ref[...], xt[_NCAT:, :], preferred_element_type=jnp.float32)
    o_ref[...] = emb + sig + b_ref[...]


@jax.jit
def kernel(x, emb_0, emb_1, emb_2, emb_3, emb_4, emb_5, emb_6, emb_7, emb_8, W, b):
    n = x.shape[0]
    xt = x.T
    tables = [emb_0, emb_1, emb_2, emb_3, emb_4, emb_5, emb_6, emb_7, emb_8]
    et = jnp.concatenate(
        tables + [jnp.zeros((_VPAD - _TOT, _EMB), jnp.float32)], axis=0
    ).T
    w1t = W[:_EMB, :].T
    w2t = W[_EMB:, :].T
    st = jnp.asarray(_S)
    tt = jnp.asarray(_T)
    b2 = b.reshape(_EMB, 1)
    grid = (pl.cdiv(n, _BLOCK),)
    outt = pl.pallas_call(
        _body,
        grid=grid,
        in_specs=[
            pl.BlockSpec((_NCOL, _BLOCK), lambda i: (0, i)),
            pl.BlockSpec((_VPAD, _NCAT), lambda i: (0, 0)),
            pl.BlockSpec((_VPAD, 1), lambda i: (0, 0)),
            pl.BlockSpec((_EMB, _VPAD), lambda i: (0, 0)),
            pl.BlockSpec((_EMB, _EMB), lambda i: (0, 0)),
            pl.BlockSpec((_EMB, _NCOL - _NCAT), lambda i: (0, 0)),
            pl.BlockSpec((_EMB, 1), lambda i: (0, 0)),
        ],
        out_specs=pl.BlockSpec((_EMB, _BLOCK), lambda i: (0, i)),
        out_shape=jax.ShapeDtypeStruct((_EMB, n), jnp.float32),
        compiler_params=pltpu.CompilerParams(
            dimension_semantics=("arbitrary",),
        ),
    )(xt, st, tt, et, w1t, w2t, b2)
    return outt.T


# speedup vs baseline: 42.7122x; 1.0062x over previous
import numpy as np
import jax
import jax.numpy as jnp
from jax.experimental import pallas as pl
from jax.experimental.pallas import tpu as pltpu

_FEATURE_DIMS = [119, 5, 12, 12, 10, 6, 6, 2, 2]
_OFFS = [0, 119, 124, 136, 148, 158, 164, 170, 172]
_NCAT = 9
_TOT = 174
_VPAD = 256
_EMB = 64
_NCOL = _NCAT + 32
_BLOCK = 8192

_S = np.zeros((_VPAD, _NCAT), np.float32)
_T = np.full((_VPAD, 1), -1.0, np.float32)
for _i in range(_NCAT):
    _lo = _OFFS[_i]
    _hi = _lo + _FEATURE_DIMS[_i]
    _S[_lo:_hi, _i] = 1.0
    _T[_lo:_hi, 0] = np.arange(_hi - _lo, dtype=np.float32)


def _body(xt_ref, st_ref, tt_ref, et_ref, w1t_ref, w2t_ref, b_ref, o_ref):
    xt = xt_ref[...]
    cat = jnp.trunc(xt[:_NCAT, :]).astype(jnp.bfloat16)
    c = jnp.dot(st_ref[...], cat, preferred_element_type=jnp.float32)
    oh = (c == tt_ref[...]).astype(jnp.bfloat16)
    a = jnp.dot(w1t_ref[...], et_ref[...], preferred_element_type=jnp.float32)
    emb = jnp.dot(a.astype(jnp.bfloat16), oh, preferred_element_type=jnp.float32)
    sig = jnp.dot(w2t_ref[...], xt[_NCAT:, :].astype(jnp.bfloat16),
                  preferred_element_type=jnp.float32)
    o_ref[...] = emb + sig + b_ref[...]


@jax.jit
def kernel(x, emb_0, emb_1, emb_2, emb_3, emb_4, emb_5, emb_6, emb_7, emb_8, W, b):
    n = x.shape[0]
    xt = x.T
    tables = [emb_0, emb_1, emb_2, emb_3, emb_4, emb_5, emb_6, emb_7, emb_8]
    et = jnp.concatenate(
        tables + [jnp.zeros((_VPAD - _TOT, _EMB), jnp.float32)], axis=0
    ).T
    w1t = W[:_EMB, :].T
    w2t = W[_EMB:, :].T
    st = jnp.asarray(_S, dtype=jnp.bfloat16)
    tt = jnp.asarray(_T)
    w2t = w2t.astype(jnp.bfloat16)
    b2 = b.reshape(_EMB, 1)
    grid = (pl.cdiv(n, _BLOCK),)
    outt = pl.pallas_call(
        _body,
        grid=grid,
        in_specs=[
            pl.BlockSpec((_NCOL, _BLOCK), lambda i: (0, i)),
            pl.BlockSpec((_VPAD, _NCAT), lambda i: (0, 0)),
            pl.BlockSpec((_VPAD, 1), lambda i: (0, 0)),
            pl.BlockSpec((_EMB, _VPAD), lambda i: (0, 0)),
            pl.BlockSpec((_EMB, _EMB), lambda i: (0, 0)),
            pl.BlockSpec((_EMB, _NCOL - _NCAT), lambda i: (0, 0)),
            pl.BlockSpec((_EMB, 1), lambda i: (0, 0)),
        ],
        out_specs=pl.BlockSpec((_EMB, _BLOCK), lambda i: (0, i)),
        out_shape=jax.ShapeDtypeStruct((_EMB, n), jnp.float32),
        compiler_params=pltpu.CompilerParams(
            dimension_semantics=("arbitrary",),
        ),
    )(xt, st, tt, et, w1t, w2t, b2)
    return outt.T


# speedup vs baseline: 47.7759x; 1.1186x over previous
import numpy as np
import jax
import jax.numpy as jnp
from jax.experimental import pallas as pl
from jax.experimental.pallas import tpu as pltpu

_FEATURE_DIMS = [119, 5, 12, 12, 10, 6, 6, 2, 2]
_OFFS = [0, 119, 124, 136, 148, 158, 164, 170, 172]
_NCAT = 9
_TOT = 174
_VPAD = 176
_EMB = 64
_NCOL = _NCAT + 32
_BLOCK = 8192

_S = np.zeros((_VPAD, _NCAT), np.float32)
_T = np.full((_VPAD, 1), -1.0, np.float32)
for _i in range(_NCAT):
    _lo = _OFFS[_i]
    _hi = _lo + _FEATURE_DIMS[_i]
    _S[_lo:_hi, _i] = 1.0
    _T[_lo:_hi, 0] = np.arange(_hi - _lo, dtype=np.float32)


def _body(xt_ref, st_ref, tt_ref, et_ref, w1t_ref, w2t_ref, b_ref, o_ref):
    xt = xt_ref[...]
    cat = jnp.trunc(xt[:_NCAT, :]).astype(jnp.bfloat16)
    c = jnp.dot(st_ref[...], cat, preferred_element_type=jnp.float32)
    oh = (c == tt_ref[...]).astype(jnp.bfloat16)
    a = jnp.dot(w1t_ref[...], et_ref[...], preferred_element_type=jnp.float32)
    emb = jnp.dot(a.astype(jnp.bfloat16), oh, preferred_element_type=jnp.float32)
    sig = jnp.dot(w2t_ref[...], xt[_NCAT:, :].astype(jnp.bfloat16),
                  preferred_element_type=jnp.float32)
    o_ref[...] = emb + sig + b_ref[...]


@jax.jit
def kernel(x, emb_0, emb_1, emb_2, emb_3, emb_4, emb_5, emb_6, emb_7, emb_8, W, b):
    n = x.shape[0]
    xt = x.T
    tables = [emb_0, emb_1, emb_2, emb_3, emb_4, emb_5, emb_6, emb_7, emb_8]
    et = jnp.concatenate(
        tables + [jnp.zeros((_VPAD - _TOT, _EMB), jnp.float32)], axis=0
    ).T
    w1t = W[:_EMB, :].T
    w2t = W[_EMB:, :].T
    st = jnp.asarray(_S, dtype=jnp.bfloat16)
    tt = jnp.asarray(_T)
    w2t = w2t.astype(jnp.bfloat16)
    b2 = b.reshape(_EMB, 1)
    grid = (pl.cdiv(n, _BLOCK),)
    outt = pl.pallas_call(
        _body,
        grid=grid,
        in_specs=[
            pl.BlockSpec((_NCOL, _BLOCK), lambda i: (0, i)),
            pl.BlockSpec((_VPAD, _NCAT), lambda i: (0, 0)),
            pl.BlockSpec((_VPAD, 1), lambda i: (0, 0)),
            pl.BlockSpec((_EMB, _VPAD), lambda i: (0, 0)),
            pl.BlockSpec((_EMB, _EMB), lambda i: (0, 0)),
            pl.BlockSpec((_EMB, _NCOL - _NCAT), lambda i: (0, 0)),
            pl.BlockSpec((_EMB, 1), lambda i: (0, 0)),
        ],
        out_specs=pl.BlockSpec((_EMB, _BLOCK), lambda i: (0, i)),
        out_shape=jax.ShapeDtypeStruct((_EMB, n), jnp.float32),
        compiler_params=pltpu.CompilerParams(
            dimension_semantics=("arbitrary",),
        ),
    )(xt, st, tt, et, w1t, w2t, b2)
    return outt.T


# speedup vs baseline: 49.0492x; 1.0266x over previous
import numpy as np
import jax
import jax.numpy as jnp
from jax.experimental import pallas as pl
from jax.experimental.pallas import tpu as pltpu

_FEATURE_DIMS = [119, 5, 12, 12, 10, 6, 6, 2, 2]
_OFFS = [0, 119, 124, 136, 148, 158, 164, 170, 172]
_NCAT = 9
_TOT = 174
_VPAD = 176
_EMB = 64
_NCOL = _NCAT + 32
_BLOCK = 16384

_S = np.zeros((_VPAD, _NCAT), np.float32)
_T = np.full((_VPAD, 1), -1.0, np.float32)
for _i in range(_NCAT):
    _lo = _OFFS[_i]
    _hi = _lo + _FEATURE_DIMS[_i]
    _S[_lo:_hi, _i] = 1.0
    _T[_lo:_hi, 0] = np.arange(_hi - _lo, dtype=np.float32)


def _body(xt_ref, st_ref, tt_ref, et_ref, w1t_ref, w2t_ref, b_ref, o_ref):
    xt = xt_ref[...]
    cat = jnp.trunc(xt[:_NCAT, :]).astype(jnp.bfloat16)
    c = jnp.dot(st_ref[...], cat, preferred_element_type=jnp.float32)
    oh = (c == tt_ref[...]).astype(jnp.bfloat16)
    a = jnp.dot(w1t_ref[...], et_ref[...], preferred_element_type=jnp.float32)
    emb = jnp.dot(a.astype(jnp.bfloat16), oh, preferred_element_type=jnp.float32)
    sig = jnp.dot(w2t_ref[...], xt[_NCAT:, :].astype(jnp.bfloat16),
                  preferred_element_type=jnp.float32)
    o_ref[...] = emb + sig + b_ref[...]


@jax.jit
def kernel(x, emb_0, emb_1, emb_2, emb_3, emb_4, emb_5, emb_6, emb_7, emb_8, W, b):
    n = x.shape[0]
    xt = x.T
    tables = [emb_0, emb_1, emb_2, emb_3, emb_4, emb_5, emb_6, emb_7, emb_8]
    et = jnp.concatenate(
        tables + [jnp.zeros((_VPAD - _TOT, _EMB), jnp.float32)], axis=0
    ).T
    w1t = W[:_EMB, :].T
    w2t = W[_EMB:, :].T
    st = jnp.asarray(_S, dtype=jnp.bfloat16)
    tt = jnp.asarray(_T)
    w2t = w2t.astype(jnp.bfloat16)
    b2 = b.reshape(_EMB, 1)
    grid = (pl.cdiv(n, _BLOCK),)
    outt = pl.pallas_call(
        _body,
        grid=grid,
        in_specs=[
            pl.BlockSpec((_NCOL, _BLOCK), lambda i: (0, i)),
            pl.BlockSpec((_VPAD, _NCAT), lambda i: (0, 0)),
            pl.BlockSpec((_VPAD, 1), lambda i: (0, 0)),
            pl.BlockSpec((_EMB, _VPAD), lambda i: (0, 0)),
            pl.BlockSpec((_EMB, _EMB), lambda i: (0, 0)),
            pl.BlockSpec((_EMB, _NCOL - _NCAT), lambda i: (0, 0)),
            pl.BlockSpec((_EMB, 1), lambda i: (0, 0)),
        ],
        out_specs=pl.BlockSpec((_EMB, _BLOCK), lambda i: (0, i)),
        out_shape=jax.ShapeDtypeStruct((_EMB, n), jnp.float32),
        compiler_params=pltpu.CompilerParams(
            dimension_semantics=("arbitrary",),
        ),
    )(xt, st, tt, et, w1t, w2t, b2)
    return outt.T


# speedup vs baseline: 50.5416x; 1.0304x over previous
import numpy as np
import jax
import jax.numpy as jnp
from jax.experimental import pallas as pl
from jax.experimental.pallas import tpu as pltpu

_FEATURE_DIMS = [119, 5, 12, 12, 10, 6, 6, 2, 2]
_OFFS = [0, 119, 124, 136, 148, 158, 164, 170, 172]
_NCAT = 9
_TOT = 174
_VPAD = 176
_EMB = 64
_NCOL = _NCAT + 32
_BLOCK = 25088

_S = np.zeros((_VPAD, _NCAT), np.float32)
_T = np.full((_VPAD, 1), -1.0, np.float32)
for _i in range(_NCAT):
    _lo = _OFFS[_i]
    _hi = _lo + _FEATURE_DIMS[_i]
    _S[_lo:_hi, _i] = 1.0
    _T[_lo:_hi, 0] = np.arange(_hi - _lo, dtype=np.float32)


def _body(xt_ref, st_ref, tt_ref, et_ref, w1t_ref, w2t_ref, b_ref, o_ref):
    xt = xt_ref[...]
    cat = jnp.trunc(xt[:_NCAT, :]).astype(jnp.bfloat16)
    c = jnp.dot(st_ref[...], cat, preferred_element_type=jnp.float32)
    oh = (c == tt_ref[...]).astype(jnp.bfloat16)
    a = jnp.dot(w1t_ref[...], et_ref[...], preferred_element_type=jnp.float32)
    emb = jnp.dot(a.astype(jnp.bfloat16), oh, preferred_element_type=jnp.float32)
    sig = jnp.dot(w2t_ref[...], xt[_NCAT:, :].astype(jnp.bfloat16),
                  preferred_element_type=jnp.float32)
    o_ref[...] = emb + sig + b_ref[...]


@jax.jit
def kernel(x, emb_0, emb_1, emb_2, emb_3, emb_4, emb_5, emb_6, emb_7, emb_8, W, b):
    n = x.shape[0]
    xt = x.T
    tables = [emb_0, emb_1, emb_2, emb_3, emb_4, emb_5, emb_6, emb_7, emb_8]
    et = jnp.concatenate(
        tables + [jnp.zeros((_VPAD - _TOT, _EMB), jnp.float32)], axis=0
    ).T
    w1t = W[:_EMB, :].T
    w2t = W[_EMB:, :].T
    st = jnp.asarray(_S, dtype=jnp.bfloat16)
    tt = jnp.asarray(_T)
    w2t = w2t.astype(jnp.bfloat16)
    b2 = b.reshape(_EMB, 1)
    grid = (pl.cdiv(n, _BLOCK),)
    outt = pl.pallas_call(
        _body,
        grid=grid,
        in_specs=[
            pl.BlockSpec((_NCOL, _BLOCK), lambda i: (0, i)),
            pl.BlockSpec((_VPAD, _NCAT), lambda i: (0, 0)),
            pl.BlockSpec((_VPAD, 1), lambda i: (0, 0)),
            pl.BlockSpec((_EMB, _VPAD), lambda i: (0, 0)),
            pl.BlockSpec((_EMB, _EMB), lambda i: (0, 0)),
            pl.BlockSpec((_EMB, _NCOL - _NCAT), lambda i: (0, 0)),
            pl.BlockSpec((_EMB, 1), lambda i: (0, 0)),
        ],
        out_specs=pl.BlockSpec((_EMB, _BLOCK), lambda i: (0, i)),
        out_shape=jax.ShapeDtypeStruct((_EMB, n), jnp.float32),
        compiler_params=pltpu.CompilerParams(
            dimension_semantics=("arbitrary",),
        ),
    )(xt, st, tt, et, w1t, w2t, b2)
    return outt.T


# speedup vs baseline: 51.6363x; 1.0217x over previous
import numpy as np
import jax
import jax.numpy as jnp
from jax.experimental import pallas as pl
from jax.experimental.pallas import tpu as pltpu

_FEATURE_DIMS = [119, 5, 12, 12, 10, 6, 6, 2, 2]
_OFFS = [0, 119, 124, 136, 148, 158, 164, 170, 172]
_NCAT = 9
_TOT = 174
_VPAD = 176
_EMB = 64
_NCOL = _NCAT + 32
_BLOCK = 33408

_S = np.zeros((_VPAD, _NCAT), np.float32)
_T = np.full((_VPAD, 1), -1.0, np.float32)
for _i in range(_NCAT):
    _lo = _OFFS[_i]
    _hi = _lo + _FEATURE_DIMS[_i]
    _S[_lo:_hi, _i] = 1.0
    _T[_lo:_hi, 0] = np.arange(_hi - _lo, dtype=np.float32)


def _body(xt_ref, st_ref, tt_ref, et_ref, w1t_ref, w2t_ref, b_ref, o_ref):
    xt = xt_ref[...]
    cat = jnp.trunc(xt[:_NCAT, :]).astype(jnp.bfloat16)
    c = jnp.dot(st_ref[...], cat, preferred_element_type=jnp.float32)
    oh = (c == tt_ref[...]).astype(jnp.bfloat16)
    a = jnp.dot(w1t_ref[...], et_ref[...], preferred_element_type=jnp.float32)
    emb = jnp.dot(a.astype(jnp.bfloat16), oh, preferred_element_type=jnp.float32)
    sig = jnp.dot(w2t_ref[...], xt[_NCAT:, :].astype(jnp.bfloat16),
                  preferred_element_type=jnp.float32)
    o_ref[...] = emb + sig + b_ref[...]


@jax.jit
def kernel(x, emb_0, emb_1, emb_2, emb_3, emb_4, emb_5, emb_6, emb_7, emb_8, W, b):
    n = x.shape[0]
    xt = x.T
    tables = [emb_0, emb_1, emb_2, emb_3, emb_4, emb_5, emb_6, emb_7, emb_8]
    et = jnp.concatenate(
        tables + [jnp.zeros((_VPAD - _TOT, _EMB), jnp.float32)], axis=0
    ).T
    w1t = W[:_EMB, :].T
    w2t = W[_EMB:, :].T
    st = jnp.asarray(_S, dtype=jnp.bfloat16)
    tt = jnp.asarray(_T)
    w2t = w2t.astype(jnp.bfloat16)
    b2 = b.reshape(_EMB, 1)
    grid = (pl.cdiv(n, _BLOCK),)
    outt = pl.pallas_call(
        _body,
        grid=grid,
        in_specs=[
            pl.BlockSpec((_NCOL, _BLOCK), lambda i: (0, i)),
            pl.BlockSpec((_VPAD, _NCAT), lambda i: (0, 0)),
            pl.BlockSpec((_VPAD, 1), lambda i: (0, 0)),
            pl.BlockSpec((_EMB, _VPAD), lambda i: (0, 0)),
            pl.BlockSpec((_EMB, _EMB), lambda i: (0, 0)),
            pl.BlockSpec((_EMB, _NCOL - _NCAT), lambda i: (0, 0)),
            pl.BlockSpec((_EMB, 1), lambda i: (0, 0)),
        ],
        out_specs=pl.BlockSpec((_EMB, _BLOCK), lambda i: (0, i)),
        out_shape=jax.ShapeDtypeStruct((_EMB, n), jnp.float32),
        compiler_params=pltpu.CompilerParams(
            dimension_semantics=("arbitrary",),
        ),
    )(xt, st, tt, et, w1t, w2t, b2)
    return outt.T
